# baseline (speedup 1.0000x reference)
import jax
import jax.numpy as jnp
from jax.experimental import pallas as pl

_B = 64
_T = 50
_H = 128
_RH = 128
_C = 10


def _gru_fc_kernel(seq_ref, w_ih_ref, w_hh_ref, brzn_ref, bhn_ref,
                   fc_w_ref, fc_b_ref, out_ref):
    w_ih = w_ih_ref[:, :].astype(jnp.bfloat16)
    w_hh = w_hh_ref[:, :].astype(jnp.bfloat16)
    brzn = brzn_ref[:, :]
    bhn = bhn_ref[:, :]

    ht = jnp.zeros((_RH, _B), jnp.float32)
    for t in range(_T):
        x_t = seq_ref[:, t * _H:(t + 1) * _H].astype(jnp.bfloat16)
        g = jax.lax.dot_general(
            w_ih, x_t, (((1,), (1,)), ((), ())),
            preferred_element_type=jnp.float32) + brzn
        gh = jax.lax.dot_general(
            w_hh, ht.astype(jnp.bfloat16), (((1,), (0,)), ((), ())),
            preferred_element_type=jnp.float32)
        r = 0.5 + 0.5 * jnp.tanh(0.5 * (g[:_RH, :] + gh[:_RH, :]))
        z = 0.5 + 0.5 * jnp.tanh(0.5 * (g[_RH:2 * _RH, :] + gh[_RH:2 * _RH, :]))
        n = jnp.tanh(g[2 * _RH:, :] + r * (gh[2 * _RH:, :] + bhn))
        ht = n + z * (ht - n)

    out_ref[:, :] = jax.lax.dot_general(
        fc_w_ref[:, :], ht, (((1,), (0,)), ((), ())),
        preferred_element_type=jnp.float32) + fc_b_ref[:, :]


def kernel(x, edge_index, sequences, W1, b1, W2, b2,
           w_ih, w_hh, b_ih, b_hh, fc_W, fc_b):
    seqflat = sequences.reshape(_B, _T * _H)
    brzn = (b_ih + jnp.concatenate(
        [b_hh[:2 * _RH], jnp.zeros((_RH,), jnp.float32)])).reshape(-1, 1)
    bhn = b_hh[2 * _RH:].reshape(-1, 1)
    logits_t = pl.pallas_call(
        _gru_fc_kernel,
        out_shape=jax.ShapeDtypeStruct((_C, _B), jnp.float32),
    )(seqflat, w_ih, w_hh, brzn, bhn, fc_W, fc_b.reshape(-1, 1))
    return logits_t.T


# speedup vs baseline: 1.1411x; 1.1411x over previous
import jax
import jax.numpy as jnp
from jax.experimental import pallas as pl
from jax.experimental.pallas import tpu as pltpu

_B = 64
_T = 50
_H = 128
_RH = 128
_C = 10


def _dot_t(a, b):
    return jax.lax.dot_general(a, b, (((1,), (1,)), ((), ())),
                               preferred_element_type=jnp.float32)


def _gru_fc_kernel(seq_ref, w_ih_ref, w_hh_ref, brzn_ref, bhn_ref,
                   fc_w_ref, fc_b_ref, out_ref, xtm_ref, gall_ref):
    w_hh = w_hh_ref[:, :].astype(jnp.bfloat16)
    brzn = brzn_ref[:, :]
    bhn = bhn_ref[:, :]

    for t in range(_T):
        xtm_ref[t * _B:(t + 1) * _B, :] = (
            seq_ref[:, t * _H:(t + 1) * _H].astype(jnp.bfloat16))

    gall_ref[:, :] = _dot_t(
        xtm_ref[:, :], w_ih_ref[:, :].astype(jnp.bfloat16)) + brzn

    h = jnp.zeros((_B, _RH), jnp.float32)
    for t in range(_T):
        g = gall_ref[t * _B:(t + 1) * _B, :]
        gh = _dot_t(h.astype(jnp.bfloat16), w_hh)
        r = 0.5 + 0.5 * jnp.tanh(0.5 * (g[:, :_RH] + gh[:, :_RH]))
        z = 0.5 + 0.5 * jnp.tanh(0.5 * (g[:, _RH:2 * _RH] + gh[:, _RH:2 * _RH]))
        n = jnp.tanh(g[:, 2 * _RH:] + r * (gh[:, 2 * _RH:] + bhn))
        h = n + z * (h - n)

    out_ref[:, :] = _dot_t(h, fc_w_ref[:, :]) + fc_b_ref[:, :]


def kernel(x, edge_index, sequences, W1, b1, W2, b2,
           w_ih, w_hh, b_ih, b_hh, fc_W, fc_b):
    seqflat = sequences.reshape(_B, _T * _H)
    brzn = (b_ih + jnp.concatenate(
        [b_hh[:2 * _RH], jnp.zeros((_RH,), jnp.float32)])).reshape(1, -1)
    bhn = b_hh[2 * _RH:].reshape(1, -1)
    return pl.pallas_call(
        _gru_fc_kernel,
        out_shape=jax.ShapeDtypeStruct((_B, _C), jnp.float32),
        scratch_shapes=[
            pltpu.VMEM((_T * _B, _H), jnp.bfloat16),
            pltpu.VMEM((_T * _B, 3 * _RH), jnp.float32),
        ],
    )(seqflat, w_ih, w_hh, brzn, bhn, fc_W, fc_b.reshape(1, -1))
